# baseline (speedup 1.0000x reference)
import functools

import jax
import jax.numpy as jnp
from jax import lax
from jax.experimental import pallas as pl
from jax.experimental.pallas import tpu as pltpu
from jax.experimental.pallas import tpu_sc as plsc

N_NODES = 1000000
D = 16
B = 16384
L = 16
CHUNK = 128


def kernel(src, dst, h_output, node_biases, mu):
    info = plsc.get_sparse_core_info()
    NC, NS = info.num_cores, info.num_subcores
    NW = NC * NS
    BPW = B // NW

    mesh = plsc.VectorSubcoreMesh(core_axis_name="c", subcore_axis_name="s")

    @functools.partial(
        pl.kernel,
        out_type=jax.ShapeDtypeStruct((B,), jnp.float32),
        mesh=mesh,
        compiler_params=pltpu.CompilerParams(
            needs_layout_passes=False, use_tc_tiling_on_sc=False),
        scratch_types=[
            pltpu.VMEM((BPW,), jnp.int32),
            pltpu.VMEM((BPW,), jnp.int32),
            pltpu.VMEM((BPW,), jnp.int32),
            pltpu.VMEM((BPW,), jnp.int32),
            pltpu.VMEM((BPW, D), jnp.float32),
            pltpu.VMEM((BPW, D), jnp.float32),
            pltpu.VMEM((BPW,), jnp.float32),
            pltpu.VMEM((BPW,), jnp.float32),
            pltpu.VMEM((BPW,), jnp.float32),
            pltpu.VMEM((L,), jnp.float32),
            pltpu.SemaphoreType.DMA,
        ],
    )
    def body(src_hbm, dst_hbm, h_hbm, nb_hbm, mu_hbm, out_hbm,
             sidx, didx, sp1, dp1, hs, hd, bs, bd, ob, muv, sem):
        wid = lax.axis_index("s") * NC + lax.axis_index("c")
        base = wid * BPW

        pltpu.sync_copy(src_hbm.at[pl.ds(base, BPW)], sidx)
        pltpu.sync_copy(dst_hbm.at[pl.ds(base, BPW)], didx)
        pltpu.sync_copy(mu_hbm, muv)

        copies = []
        for c in range(BPW // CHUNK):
            o = c * CHUNK
            copies.append(pltpu.async_copy(
                h_hbm.at[sidx.at[pl.ds(o, CHUNK)]],
                hs.at[pl.ds(o, CHUNK)], sem))
            copies.append(pltpu.async_copy(
                h_hbm.at[didx.at[pl.ds(o, CHUNK)]],
                hd.at[pl.ds(o, CHUNK)], sem))

        def addone(i, _):
            sl = pl.ds(i * L, L)
            sp1[sl] = sidx[sl] + 1
            dp1[sl] = didx[sl] + 1
            return 0
        lax.fori_loop(0, BPW // L, addone, 0)

        for c in range(BPW // CHUNK):
            o = c * CHUNK
            copies.append(pltpu.async_copy(
                nb_hbm.at[sp1.at[pl.ds(o, CHUNK)]],
                bs.at[pl.ds(o, CHUNK)], sem))
            copies.append(pltpu.async_copy(
                nb_hbm.at[dp1.at[pl.ds(o, CHUNK)]],
                bd.at[pl.ds(o, CHUNK)], sem))

        for cp in copies:
            cp.wait()

        mu0 = muv[...][0]

        lane = lax.iota(jnp.int32, L)

        def chunk_body(cidx, _):
            rb = cidx * L
            acc = bs[pl.ds(rb, L)] + bd[pl.ds(rb, L)] + mu0
            for i in range(L):
                prod = hs[rb + i, :] * hd[rb + i, :]
                rowsum = jnp.sum(prod)
                acc = acc + jnp.where(lane == i, rowsum, 0.0)
            ob[pl.ds(rb, L)] = acc
            return 0
        lax.fori_loop(0, BPW // L, chunk_body, 0)

        pltpu.sync_copy(ob, out_hbm.at[pl.ds(base, BPW)])

    mu16 = jnp.broadcast_to(mu, (L,))
    return body(src, dst, h_output, node_biases, mu16)
